# Initial kernel scaffold; baseline (speedup 1.0000x reference)
import jax
import jax.numpy as jnp
from jax.experimental import pallas as pl


def kernel(x, We, be, Wq, Wk, Wv, Wck, Wcv, Wg, bg, Wo, bo, ln1_g, ln1_b, Wt1, bt1, Wt2, bt2, ln2_g, ln2_b, Wf1, bf1, Wf2, bf2, Wh1, bh1, Wh2, bh2):
    raise NotImplementedError("write your pallas kernel here")


# speedup vs baseline: 1.1294x; 1.1294x over previous
import jax
import jax.numpy as jnp
from jax.experimental import pallas as pl

B, F, D, H, DH = 256, 128, 64, 3, 16
CBS, SBS, NSEL, WIN = 16, 16, 2, 8
INNER = H * DH
NB = F // CBS
BT = 8
SCALE = DH ** -0.5
NEG = -1e30


def _ln(x, g, b):
    m = jnp.mean(x, axis=-1, keepdims=True)
    v = jnp.mean((x - m) * (x - m), axis=-1, keepdims=True)
    return (x - m) * jax.lax.rsqrt(v + 1e-5) * g + b


def _msoftmax(s, mask):
    s = jnp.where(mask, s, NEG)
    m = jnp.max(s, axis=-1, keepdims=True)
    e = jnp.exp(s - m)
    return e / jnp.sum(e, axis=-1, keepdims=True)


def _softmax(s):
    m = jnp.max(s, axis=-1, keepdims=True)
    e = jnp.exp(s - m)
    return e / jnp.sum(e, axis=-1, keepdims=True)


def _dot_t(a, b):
    return jax.lax.dot_general(a, b, (((1,), (1,)), ((), ())))


def _fwd(x_ref, We_ref, be_ref, Wq_ref, Wk_ref, Wv_ref, Wck_ref, Wcv_ref,
         Wg_ref, bg_ref, Wo_ref, bo_ref, ln1g_ref, ln1b_ref,
         Wt1_ref, bt1_ref, Wt2_ref, bt2_ref, ln2g_ref, ln2b_ref,
         Wf1_ref, bf1_ref, Wf2_ref, bf2_ref, Wh1_ref, bh1_ref,
         Wh2_ref, bh2_ref, o_ref):
    xb = x_ref[...]
    we = We_ref[...]
    be = be_ref[...]
    embf = xb * we + be

    Wq = Wq_ref[...]
    Wk = Wk_ref[...]
    Wv = Wv_ref[...]
    Wck = Wck_ref[...]
    Wcv = Wcv_ref[...]
    gf = jax.nn.sigmoid(embf @ Wg_ref[...] + bg_ref[...])
    g3 = gf.reshape(BT, F, 3 * H)

    ii = jax.lax.broadcasted_iota(jnp.int32, (F, F), 0)
    jj = jax.lax.broadcasted_iota(jnp.int32, (F, F), 1)
    band = jnp.abs(ii - jj) <= WIN
    jblk = jj // CBS
    j8 = jax.lax.broadcasted_iota(jnp.int32, (F, NB), 1)

    Wo = Wo_ref[...]
    x1f = jnp.zeros((BT * F, D), jnp.float32)
    for h in range(H):
        sl = slice(h * DH, (h + 1) * DH)
        qh = (embf @ Wq[:, sl]).reshape(BT, F, DH)
        kh = (embf @ Wk[:, sl]).reshape(BT, F, DH)
        vh = (embf @ Wv[:, sl]).reshape(BT, F, DH)
        outs = []
        for b in range(BT):
            qb, kb, vb = qh[b], kh[b], vh[b]
            kc = kb.reshape(NB, CBS, DH).mean(axis=1) @ Wck
            vc = vb.reshape(NB, CBS, DH).mean(axis=1) @ Wcv
            sc = _dot_t(qb, kc) * SCALE
            oc = _softmax(sc) @ vc
            m1 = jnp.max(sc, axis=1, keepdims=True)
            idx1 = jnp.min(jnp.where(sc == m1, j8, NB), axis=1, keepdims=True)
            sc2 = jnp.where(j8 == idx1, -3e38, sc)
            m2 = jnp.max(sc2, axis=1, keepdims=True)
            idx2 = jnp.min(jnp.where(sc2 == m2, j8, NB), axis=1, keepdims=True)
            tok = (jblk == idx1) | (jblk == idx2)
            sf = _dot_t(qb, kb) * SCALE
            os_ = _msoftmax(sf, tok) @ vb
            ow = _msoftmax(sf, band) @ vb
            gb = g3[b]
            o = (gb[:, 3 * h + 0:3 * h + 1] * oc
                 + gb[:, 3 * h + 1:3 * h + 2] * os_
                 + gb[:, 3 * h + 2:3 * h + 3] * ow)
            outs.append(o)
        outh = jnp.stack(outs).reshape(BT * F, DH)
        x1f = x1f + outh @ Wo[sl, :]
    x1 = (x1f + bo_ref[...]).reshape(BT, F, D)

    emb3 = embf.reshape(BT, F, D)
    t = _ln(emb3, ln1g_ref[...], ln1b_ref[...])
    tn = jnp.swapaxes(t, 1, 2).reshape(BT * D, F)
    tz = jax.nn.gelu(tn @ Wt1_ref[...] + bt1_ref[...]) @ Wt2_ref[...] + bt2_ref[...]
    h1 = emb3 + jnp.swapaxes(tz.reshape(BT, D, F), 1, 2)
    un = _ln(h1, ln2g_ref[...], ln2b_ref[...]).reshape(BT * F, D)
    u = jax.nn.gelu(un @ Wf1_ref[...] + bf1_ref[...]) @ Wf2_ref[...] + bf2_ref[...]
    x2 = h1 + u.reshape(BT, F, D)

    pooled = jnp.mean(x1 + x2, axis=1)
    out = jax.nn.gelu(pooled @ Wh1_ref[...] + bh1_ref[...]) @ Wh2_ref[...] + bh2_ref[...]
    o_ref[...] = out


def kernel(x, We, be, Wq, Wk, Wv, Wck, Wcv, Wg, bg, Wo, bo, ln1_g, ln1_b,
           Wt1, bt1, Wt2, bt2, ln2_g, ln2_b, Wf1, bf1, Wf2, bf2,
           Wh1, bh1, Wh2, bh2):
    args = [
        x.reshape(B * F, 1), We, be.reshape(1, D), Wq, Wk, Wv, Wck, Wcv, Wg,
        bg.reshape(1, 3 * H),
        Wo, bo.reshape(1, D), ln1_g.reshape(1, D), ln1_b.reshape(1, D),
        Wt1, bt1.reshape(1, 256), Wt2, bt2.reshape(1, F),
        ln2_g.reshape(1, D), ln2_b.reshape(1, D),
        Wf1, bf1.reshape(1, 256), Wf2, bf2.reshape(1, D),
        Wh1, bh1.reshape(1, 32), Wh2, bh2.reshape(1, 2),
    ]
    in_specs = [pl.BlockSpec((BT * F, 1), lambda i: (i, 0))]
    for a in args[1:]:
        in_specs.append(pl.BlockSpec(a.shape, lambda i: (0, 0)))
    return pl.pallas_call(
        _fwd,
        grid=(B // BT,),
        in_specs=in_specs,
        out_specs=pl.BlockSpec((BT, 2), lambda i: (i, 0)),
        out_shape=jax.ShapeDtypeStruct((B, 2), jnp.float32),
    )(*args)
